# baseline (speedup 1.0000x reference)
import functools
import math

import jax
import jax.numpy as jnp
from jax import lax
from jax.experimental import pallas as pl
from jax.experimental.pallas import tpu as pltpu
from jax.experimental.pallas import tpu_sc as plsc

VOCAB = 100000
D = 300
DP = 384
SCALE = math.sqrt(300.0)
N = 4096 * 50
NC, NS = 2, 16
NW = NC * NS
PER_W = N // NW
C = 80
CHUNKS = PER_W // C

R_BLK = 2000


def _pad_scale_body(t_ref, o_ref):
    o_ref[:, :D] = t_ref[...] * SCALE
    o_ref[:, D:] = jnp.zeros((R_BLK, DP - D), jnp.float32)


_pad_scale = pl.pallas_call(
    _pad_scale_body,
    grid=(VOCAB // R_BLK,),
    in_specs=[pl.BlockSpec((R_BLK, D), lambda i: (i, 0))],
    out_specs=pl.BlockSpec((R_BLK, DP), lambda i: (i, 0)),
    out_shape=jax.ShapeDtypeStruct((VOCAB, DP), jnp.float32),
    compiler_params=pltpu.CompilerParams(
        dimension_semantics=("parallel",)),
)

_mesh = plsc.VectorSubcoreMesh(core_axis_name="c", subcore_axis_name="s")


@functools.partial(
    pl.kernel,
    mesh=_mesh,
    out_type=jax.ShapeDtypeStruct((N, DP), jnp.float32),
    scratch_types=[
        pltpu.VMEM((PER_W,), jnp.int32),
        pltpu.VMEM((C, DP), jnp.float32),
        pltpu.VMEM((C, DP), jnp.float32),
        pltpu.VMEM((C, DP), jnp.float32),
        pltpu.SemaphoreType.DMA,
        pltpu.SemaphoreType.DMA,
        pltpu.SemaphoreType.DMA,
        pltpu.SemaphoreType.DMA,
        pltpu.SemaphoreType.DMA,
        pltpu.SemaphoreType.DMA,
    ],
)
def _emb_lookup(idx_hbm, table_hbm, out_hbm, idx_all,
                buf0, buf1, buf2, g0, g1, g2, w0, w1, w2):
    wid = lax.axis_index("s") * NC + lax.axis_index("c")
    base0 = wid * PER_W

    bufs = (buf0, buf1, buf2)
    gsem = (g0, g1, g2)
    wsem = (w0, w1, w2)

    def fire_gather(k):
        pltpu.async_copy(
            table_hbm.at[idx_all.at[pl.ds(k * C, C)]],
            bufs[k % 3], gsem[k % 3])

    def wait_gather(k):
        pltpu.make_async_copy(
            table_hbm.at[pl.ds(0, C)], bufs[k % 3], gsem[k % 3]).wait()

    def fire_write(k):
        pltpu.async_copy(
            bufs[k % 3], out_hbm.at[pl.ds(base0 + k * C, C)], wsem[k % 3])

    def wait_write(k):
        pltpu.make_async_copy(
            bufs[k % 3], out_hbm.at[pl.ds(base0 + k * C, C)],
            wsem[k % 3]).wait()

    pltpu.sync_copy(idx_hbm.at[pl.ds(base0, PER_W)], idx_all)
    fire_gather(0)
    fire_gather(1)
    for k in range(CHUNKS):
        if k + 2 < CHUNKS:
            if k - 1 >= 0:
                wait_write(k - 1)
            fire_gather(k + 2)
        wait_gather(k)
        fire_write(k)
    for k in range(CHUNKS - 3, CHUNKS):
        wait_write(k)


def kernel(x, table):
    xf = x.reshape(-1).astype(jnp.int32)
    out = _emb_lookup(xf, _pad_scale(table))
    return out[:, :D].reshape(x.shape + (D,))


# speedup vs baseline: 1.0091x; 1.0091x over previous
import functools
import math

import jax
import jax.numpy as jnp
from jax import lax
from jax.experimental import pallas as pl
from jax.experimental.pallas import tpu as pltpu
from jax.experimental.pallas import tpu_sc as plsc

VOCAB = 100000
D = 300
DP = 384
SCALE = math.sqrt(300.0)
N = 4096 * 50
NC, NS = 2, 16
NW = NC * NS
PER_W = N // NW
C = 128
CHUNKS = PER_W // C

R_BLK = 2000


def _pad_scale_body(t_ref, o_ref):
    o_ref[:, :D] = t_ref[...] * SCALE
    o_ref[:, D:] = jnp.zeros((R_BLK, DP - D), jnp.float32)


_pad_scale = pl.pallas_call(
    _pad_scale_body,
    grid=(VOCAB // R_BLK,),
    in_specs=[pl.BlockSpec((R_BLK, D), lambda i: (i, 0))],
    out_specs=pl.BlockSpec((R_BLK, DP), lambda i: (i, 0)),
    out_shape=jax.ShapeDtypeStruct((VOCAB, DP), jnp.float32),
    compiler_params=pltpu.CompilerParams(
        dimension_semantics=("parallel",)),
)

_mesh = plsc.VectorSubcoreMesh(core_axis_name="c", subcore_axis_name="s")


@functools.partial(
    pl.kernel,
    mesh=_mesh,
    out_type=jax.ShapeDtypeStruct((N, DP), jnp.float32),
    scratch_types=[
        pltpu.VMEM((PER_W,), jnp.int32),
        pltpu.VMEM((C, DP), jnp.float32),
        pltpu.VMEM((C, DP), jnp.float32),
        pltpu.SemaphoreType.DMA,
        pltpu.SemaphoreType.DMA,
    ],
)
def _emb_lookup(idx_hbm, table_hbm, out_hbm, idx_all, buf0, buf1, sem0, sem1):
    wid = lax.axis_index("s") * NC + lax.axis_index("c")
    base0 = wid * PER_W

    pltpu.sync_copy(idx_hbm.at[pl.ds(base0, PER_W)], idx_all)
    pltpu.async_copy(table_hbm.at[idx_all.at[pl.ds(0, C)]], buf0, sem0)

    def chunk_body(k2, carry):
        k = 2 * k2
        pltpu.async_copy(
            table_hbm.at[idx_all.at[pl.ds((k + 1) * C, C)]], buf1, sem1)
        pltpu.make_async_copy(table_hbm.at[pl.ds(0, C)], buf0, sem0).wait()
        pltpu.sync_copy(buf0, out_hbm.at[pl.ds(base0 + k * C, C)])

        @pl.when(k2 < CHUNKS // 2 - 1)
        def _():
            pltpu.async_copy(
                table_hbm.at[idx_all.at[pl.ds((k + 2) * C, C)]], buf0, sem0)

        pltpu.make_async_copy(table_hbm.at[pl.ds(0, C)], buf1, sem1).wait()
        pltpu.sync_copy(buf1, out_hbm.at[pl.ds(base0 + (k + 1) * C, C)])
        return carry

    lax.fori_loop(0, CHUNKS // 2, chunk_body, 0)


def kernel(x, table):
    xf = x.reshape(-1).astype(jnp.int32)
    out = _emb_lookup(xf, _pad_scale(table))
    return out[:, :D].reshape(x.shape + (D,))
